# baseline (speedup 1.0000x reference)
import functools

import jax
import jax.numpy as jnp
from jax import lax
from jax.experimental import pallas as pl
from jax.experimental.pallas import tpu as pltpu
from jax.experimental.pallas import tpu_sc as plsc

C = 256
NBOX = 256
PH = 7
PW = 7
SR = 2
BINS = NBOX * PH * PW
NW = 32
ROWS_PER_W = BINS // NW
K = 16
CCHUNKS = C // 16

_SCALES = (0.25, 0.125, 0.0625, 0.03125)
_HS = (200, 100, 50, 25)
_WS = (200, 100, 50, 25)
_BASES = (0, 41600, 54400, 56900)
_TOTAL_ROWS = 57600


def _precompute(boxes):
    f32 = jnp.float32
    i32 = jnp.int32
    area = (boxes[:, 2] - boxes[:, 0]) * (boxes[:, 3] - boxes[:, 1])
    s = jnp.sqrt(area)
    target_lvls = jnp.floor(4.0 + jnp.log2(s / 224.0) + 1e-6)
    lvl = jnp.clip(target_lvls, 2.0, 5.0).astype(i32) - 2

    scale = jnp.take(jnp.array(_SCALES, f32), lvl)
    H = jnp.take(jnp.array(_HS, i32), lvl)
    W = jnp.take(jnp.array(_WS, i32), lvl)
    base = jnp.take(jnp.array(_BASES, i32), lvl)

    roi_start_w = boxes[:, 0] * scale
    roi_start_h = boxes[:, 1] * scale
    roi_end_w = boxes[:, 2] * scale
    roi_end_h = boxes[:, 3] * scale
    roi_width = jnp.maximum(roi_end_w - roi_start_w, 1.0)
    roi_height = jnp.maximum(roi_end_h - roi_start_h, 1.0)
    bin_h = roi_height / PH
    bin_w = roi_width / PW
    ph = jnp.arange(PH, dtype=f32)
    iy = jnp.arange(SR, dtype=f32)
    y = (roi_start_h[:, None, None] + ph[None, :, None] * bin_h[:, None, None]
         + (iy[None, None, :] + 0.5) * (bin_h / SR)[:, None, None])
    x = (roi_start_w[:, None, None] + ph[None, :, None] * bin_w[:, None, None]
         + (iy[None, None, :] + 0.5) * (bin_w / SR)[:, None, None])
    y = jnp.maximum(y, 0.0)
    x = jnp.maximum(x, 0.0)
    y_low = y.astype(i32)
    x_low = x.astype(i32)
    Hm1 = (H - 1)[:, None, None]
    Wm1 = (W - 1)[:, None, None]
    y_high = jnp.where(y_low >= Hm1, Hm1, y_low + 1)
    y_low = jnp.where(y_low >= Hm1, Hm1, y_low)
    x_high = jnp.where(x_low >= Wm1, Wm1, x_low + 1)
    x_low = jnp.where(x_low >= Wm1, Wm1, x_low)
    ly = y - y_low.astype(f32)
    lx = x - x_low.astype(f32)
    hy = 1.0 - ly
    hx = 1.0 - lx

    y_idx = jnp.stack([y_low, y_high], axis=-1)
    wy = jnp.stack([hy, ly], axis=-1)
    x_idx = jnp.stack([x_low, x_high], axis=-1)
    wx = jnp.stack([hx, lx], axis=-1)

    idx_full = (base[:, None, None, None, None, None, None]
                + y_idx[:, :, None, :, None, :, None] * W[:, None, None, None, None, None, None]
                + x_idx[:, None, :, None, :, None, :])
    w_full = (wy[:, :, None, :, None, :, None]
              * wx[:, None, :, None, :, None, :]) * (1.0 / (SR * SR))
    idx_full = idx_full.reshape(BINS, K)
    w_full = w_full.reshape(BINS, K).astype(f32)
    return idx_full.astype(i32), w_full


def _lane_bcast(vec, k):
    dnums = lax.GatherDimensionNumbers(
        offset_dims=(), collapsed_slice_dims=(0,), start_index_map=(0,))
    return lax.gather(vec, jnp.full((16, 1), k, jnp.int32), dnums,
                      slice_sizes=(1,),
                      mode=lax.GatherScatterMode.PROMISE_IN_BOUNDS)


HALF = ROWS_PER_W // 2


NBUF = 14
CHUNK = 56
NCHUNK = ROWS_PER_W // CHUNK
GPC = CHUNK // NBUF
NGROUPS = ROWS_PER_W // NBUF


_TBLK = 3200
_STARTS = (0, 13, 17)
_GRID = 18


def _tr_body(x0, x1, x23, o_ref):
    r = lax.broadcasted_iota(jnp.int32, (C, C), 0)
    cc = lax.broadcasted_iota(jnp.int32, (C, C), 1)
    eye = (r == cc).astype(jnp.float32)
    g = pl.program_id(0)

    def dot_t(x):
        return lax.dot_general(
            x, eye, (((0,), (0,)), ((), ())),
            preferred_element_type=jnp.float32,
            precision=lax.Precision.HIGHEST)

    @pl.when(g < _STARTS[1])
    def _l0():
        for hh in range(16):
            o_ref[pl.ds(hh * 200, 200), :] = dot_t(x0[0, :, hh, :])

    @pl.when((g >= _STARTS[1]) & (g < _STARTS[2]))
    def _l1():
        o_ref[...] = dot_t(x1[...])

    @pl.when(g >= _STARTS[2])
    def _l23():
        o_ref[...] = dot_t(x23[...])


@jax.jit
def _build_table(f0, f1, f23):
    in_specs = [
        pl.BlockSpec((1, C, 16, 200),
                     lambda g: (0, 0, jnp.clip(g, 0, 12), 0)),
        pl.BlockSpec((C, _TBLK),
                     lambda g: (0, jnp.clip(g - _STARTS[1], 0, 3))),
        pl.BlockSpec((C, _TBLK), lambda g: (0, 0)),
    ]
    return pl.pallas_call(
        _tr_body,
        grid=(_GRID,),
        in_specs=in_specs,
        out_specs=pl.BlockSpec((_TBLK, C), lambda g: (g, 0)),
        out_shape=jax.ShapeDtypeStruct((_TOTAL_ROWS, C), jnp.float32),
    )(f0, f1, f23)


BOXES_PER_W = NBOX // NW


def _sc_body(table, params, out, idx_v, w_v, gbuf, obuf, pbuf, sem, osem):
    wid = lax.axis_index("s") * 2 + lax.axis_index("c")
    base = wid * ROWS_PER_W

    kiota = lax.iota(jnp.int32, 16)
    iyf = ((kiota >> 3) & 1).astype(jnp.float32)
    ixf = ((kiota >> 2) & 1).astype(jnp.float32)
    cy1 = ((kiota >> 1) & 1) == 1
    cx1 = (kiota & 1) == 1

    pltpu.sync_copy(params.at[pl.ds(wid * BOXES_PER_W, BOXES_PER_W)], pbuf)

    for bx in range(BOXES_PER_W):
        pv = pbuf[bx]
        sh = _lane_bcast(pv, 0)
        sw = _lane_bcast(pv, 1)
        bh = _lane_bcast(pv, 2)
        bw = _lane_bcast(pv, 3)
        bhh = _lane_bcast(pv, 4)
        bwh = _lane_bcast(pv, 5)
        wstride = _lane_bcast(pv, 6).astype(jnp.int32)
        hm1 = _lane_bcast(pv, 7).astype(jnp.int32)
        wm1 = _lane_bcast(pv, 8).astype(jnp.int32)
        basei = _lane_bcast(pv, 9).astype(jnp.int32)

        def ph_loop(ph, c1):
            phf = jnp.full((16,), ph.astype(jnp.float32))
            y = sh + phf * bh + (iyf + 0.5) * bhh
            y = jnp.maximum(y, 0.0)
            yl0 = y.astype(jnp.int32)
            yh = jnp.where(yl0 >= hm1, hm1, yl0 + 1)
            yl = jnp.where(yl0 >= hm1, hm1, yl0)
            ly = y - yl.astype(jnp.float32)
            wy = jnp.where(cy1, ly, 1.0 - ly)
            ysel = jnp.where(cy1, yh, yl)
            yterm = basei + ysel * wstride

            def pw_loop(pw, c2):
                pwf = jnp.full((16,), pw.astype(jnp.float32))
                x = sw + pwf * bw + (ixf + 0.5) * bwh
                x = jnp.maximum(x, 0.0)
                xl0 = x.astype(jnp.int32)
                xh = jnp.where(xl0 >= wm1, wm1, xl0 + 1)
                xl = jnp.where(xl0 >= wm1, wm1, xl0)
                lx = x - xl.astype(jnp.float32)
                wx = jnp.where(cx1, lx, 1.0 - lx)
                xsel = jnp.where(cx1, xh, xl)
                o = (bx * (PH * PW) + ph * PW + pw) * K
                idx_v[pl.ds(o, K)] = yterm + xsel
                w_v[pl.ds(o, K)] = wy * wx * 0.25
                return c2

            lax.fori_loop(0, PW, pw_loop, 0)
            return c1

        lax.fori_loop(0, PH, ph_loop, 0)


    def issue(i, b):
        idx_vec = idx_v[pl.ds(i * K, K)]
        pltpu.async_copy(table.at[idx_vec], gbuf.at[b], sem.at[b])

    def drain(b):
        pltpu.make_async_copy(table.at[pl.ds(0, K)], gbuf.at[b],
                              sem.at[b]).wait()

    def odrain(h):
        pltpu.make_async_copy(obuf.at[h], out.at[pl.ds(base, CHUNK)],
                              osem.at[h]).wait()

    for b in range(NBUF):
        issue(b, b)

    def chunk_loop(cix, carry):
        h = lax.rem(cix, 2)

        @pl.when(cix >= 2)
        def _wait_prev_flush():
            odrain(h)

        def group(gg, carry2):
            g = cix * GPC + gg
            for b in range(NBUF):
                i = g * NBUF + b
                local = gg * NBUF + b
                drain(b)
                w_vec = w_v[pl.ds(i * K, K)]
                acc = [jnp.zeros((16,), jnp.float32) for _ in range(CCHUNKS)]
                for k in range(K):
                    wk = _lane_bcast(w_vec, k)
                    for j in range(CCHUNKS):
                        acc[j] = acc[j] + wk * gbuf[b, k, pl.ds(16 * j, 16)]
                for j in range(CCHUNKS):
                    obuf[h, local, pl.ds(16 * j, 16)] = acc[j]

                @pl.when(g < NGROUPS - 1)
                def _issue_next():
                    issue(i + NBUF, b)
            return carry2

        lax.fori_loop(0, GPC, group, 0)
        pltpu.async_copy(obuf.at[h], out.at[pl.ds(base + cix * CHUNK, CHUNK)],
                         osem.at[h])
        return carry

    lax.fori_loop(0, NCHUNK, chunk_loop, 0)
    odrain(1)
    odrain(0)


@jax.jit
def _sc_gather_interp(table, params):
    mesh = plsc.VectorSubcoreMesh(core_axis_name="c", subcore_axis_name="s")
    f = functools.partial(
        pl.kernel, mesh=mesh,
        out_type=jax.ShapeDtypeStruct((BINS, C), jnp.float32),
        scratch_types=[
            pltpu.VMEM((ROWS_PER_W * K,), jnp.int32),
            pltpu.VMEM((ROWS_PER_W * K,), jnp.float32),
            pltpu.VMEM((NBUF, K, C), jnp.float32),
            pltpu.VMEM((2, CHUNK, C), jnp.float32),
            pltpu.VMEM((BOXES_PER_W, 16), jnp.float32),
            pltpu.SemaphoreType.DMA((NBUF,)),
            pltpu.SemaphoreType.DMA((2,)),
        ],
    )(_sc_body)
    return f(table, params)


def _box_params(boxes):
    f32 = jnp.float32
    area = (boxes[:, 2] - boxes[:, 0]) * (boxes[:, 3] - boxes[:, 1])
    s = jnp.sqrt(area)
    target_lvls = jnp.floor(4.0 + jnp.log2(s / 224.0) + 1e-6)
    lvl = jnp.clip(target_lvls, 2.0, 5.0).astype(jnp.int32) - 2
    scale = jnp.take(jnp.array(_SCALES, f32), lvl)
    wf = jnp.take(jnp.array(_WS, f32), lvl)
    hf = jnp.take(jnp.array(_HS, f32), lvl)
    basef = jnp.take(jnp.array(_BASES, f32), lvl)
    start_w = boxes[:, 0] * scale
    start_h = boxes[:, 1] * scale
    end_w = boxes[:, 2] * scale
    end_h = boxes[:, 3] * scale
    roi_w = jnp.maximum(end_w - start_w, 1.0)
    roi_h = jnp.maximum(end_h - start_h, 1.0)
    bin_h = roi_h / PH
    bin_w = roi_w / PW
    z = jnp.zeros_like(scale)
    cols = [start_h, start_w, bin_h, bin_w, bin_h / SR, bin_w / SR,
            wf, hf - 1.0, wf - 1.0, basef, z, z, z, z, z, z]
    return jnp.stack(cols, axis=1)


def kernel(feat0, feat1, feat2, feat3, boxes):
    f23 = jnp.concatenate(
        [feat2.reshape(C, -1), feat3.reshape(C, -1),
         jnp.zeros((C, _TBLK - 3125), jnp.float32)], axis=1)
    table = _build_table(feat0, feat1.reshape(C, -1), f23)
    params = _box_params(boxes)
    res = _sc_gather_interp(table, params)
    return res.reshape(NBOX, PH * PW, C).transpose(0, 2, 1).reshape(NBOX, C, PH, PW)


# speedup vs baseline: 1.1488x; 1.1488x over previous
import functools

import jax
import jax.numpy as jnp
from jax import lax
from jax.experimental import pallas as pl
from jax.experimental.pallas import tpu as pltpu
from jax.experimental.pallas import tpu_sc as plsc

C = 256
NBOX = 256
PH = 7
PW = 7
SR = 2
BINS = NBOX * PH * PW
NW = 32
ROWS_PER_W = BINS // NW
K = 16
CCHUNKS = C // 16

_SCALES = (0.25, 0.125, 0.0625, 0.03125)
_HS = (200, 100, 50, 25)
_WS = (200, 100, 50, 25)
_BASES = (0, 40000, 50000, 52500)
_TOTAL_ROWS = 53125


def _precompute(boxes):
    f32 = jnp.float32
    i32 = jnp.int32
    area = (boxes[:, 2] - boxes[:, 0]) * (boxes[:, 3] - boxes[:, 1])
    s = jnp.sqrt(area)
    target_lvls = jnp.floor(4.0 + jnp.log2(s / 224.0) + 1e-6)
    lvl = jnp.clip(target_lvls, 2.0, 5.0).astype(i32) - 2

    scale = jnp.take(jnp.array(_SCALES, f32), lvl)
    H = jnp.take(jnp.array(_HS, i32), lvl)
    W = jnp.take(jnp.array(_WS, i32), lvl)
    base = jnp.take(jnp.array(_BASES, i32), lvl)

    roi_start_w = boxes[:, 0] * scale
    roi_start_h = boxes[:, 1] * scale
    roi_end_w = boxes[:, 2] * scale
    roi_end_h = boxes[:, 3] * scale
    roi_width = jnp.maximum(roi_end_w - roi_start_w, 1.0)
    roi_height = jnp.maximum(roi_end_h - roi_start_h, 1.0)
    bin_h = roi_height / PH
    bin_w = roi_width / PW
    ph = jnp.arange(PH, dtype=f32)
    iy = jnp.arange(SR, dtype=f32)
    y = (roi_start_h[:, None, None] + ph[None, :, None] * bin_h[:, None, None]
         + (iy[None, None, :] + 0.5) * (bin_h / SR)[:, None, None])
    x = (roi_start_w[:, None, None] + ph[None, :, None] * bin_w[:, None, None]
         + (iy[None, None, :] + 0.5) * (bin_w / SR)[:, None, None])
    y = jnp.maximum(y, 0.0)
    x = jnp.maximum(x, 0.0)
    y_low = y.astype(i32)
    x_low = x.astype(i32)
    Hm1 = (H - 1)[:, None, None]
    Wm1 = (W - 1)[:, None, None]
    y_high = jnp.where(y_low >= Hm1, Hm1, y_low + 1)
    y_low = jnp.where(y_low >= Hm1, Hm1, y_low)
    x_high = jnp.where(x_low >= Wm1, Wm1, x_low + 1)
    x_low = jnp.where(x_low >= Wm1, Wm1, x_low)
    ly = y - y_low.astype(f32)
    lx = x - x_low.astype(f32)
    hy = 1.0 - ly
    hx = 1.0 - lx

    y_idx = jnp.stack([y_low, y_high], axis=-1)
    wy = jnp.stack([hy, ly], axis=-1)
    x_idx = jnp.stack([x_low, x_high], axis=-1)
    wx = jnp.stack([hx, lx], axis=-1)

    idx_full = (base[:, None, None, None, None, None, None]
                + y_idx[:, :, None, :, None, :, None] * W[:, None, None, None, None, None, None]
                + x_idx[:, None, :, None, :, None, :])
    w_full = (wy[:, :, None, :, None, :, None]
              * wx[:, None, :, None, :, None, :]) * (1.0 / (SR * SR))
    idx_full = idx_full.reshape(BINS, K)
    w_full = w_full.reshape(BINS, K).astype(f32)
    return idx_full.astype(i32), w_full


def _lane_bcast(vec, k):
    dnums = lax.GatherDimensionNumbers(
        offset_dims=(), collapsed_slice_dims=(0,), start_index_map=(0,))
    return lax.gather(vec, jnp.full((16, 1), k, jnp.int32), dnums,
                      slice_sizes=(1,),
                      mode=lax.GatherScatterMode.PROMISE_IN_BOUNDS)


HALF = ROWS_PER_W // 2


NBUF = 8
CHUNK = 56
NCHUNK = ROWS_PER_W // CHUNK
GPC = CHUNK // NBUF
NGROUPS = ROWS_PER_W // NBUF


BOXES_PER_W = NBOX // NW


def _sc_body(table, params, out, idx_v, w_v, gbuf, obuf, pbuf, sem, osem):
    wid = lax.axis_index("s") * 2 + lax.axis_index("c")
    base = wid * ROWS_PER_W

    kiota = lax.iota(jnp.int32, 16)
    iyf = ((kiota >> 3) & 1).astype(jnp.float32)
    ixf = ((kiota >> 2) & 1).astype(jnp.float32)
    cy1 = ((kiota >> 1) & 1) == 1
    cx1 = (kiota & 1) == 1

    pltpu.sync_copy(params.at[pl.ds(wid * BOXES_PER_W, BOXES_PER_W)], pbuf)

    for bx in range(BOXES_PER_W):
        pv = pbuf[bx]
        sh = _lane_bcast(pv, 0)
        sw = _lane_bcast(pv, 1)
        bh = _lane_bcast(pv, 2)
        bw = _lane_bcast(pv, 3)
        bhh = _lane_bcast(pv, 4)
        bwh = _lane_bcast(pv, 5)
        wstride = _lane_bcast(pv, 6).astype(jnp.int32)
        hm1 = _lane_bcast(pv, 7).astype(jnp.int32)
        wm1 = _lane_bcast(pv, 8).astype(jnp.int32)
        basei = _lane_bcast(pv, 9).astype(jnp.int32)

        def ph_loop(ph, c1):
            phf = jnp.full((16,), ph.astype(jnp.float32))
            y = sh + phf * bh + (iyf + 0.5) * bhh
            y = jnp.maximum(y, 0.0)
            yl0 = y.astype(jnp.int32)
            yh = jnp.where(yl0 >= hm1, hm1, yl0 + 1)
            yl = jnp.where(yl0 >= hm1, hm1, yl0)
            ly = y - yl.astype(jnp.float32)
            wy = jnp.where(cy1, ly, 1.0 - ly)
            ysel = jnp.where(cy1, yh, yl)
            yterm = basei + ysel * wstride

            def pw_loop(pw, c2):
                pwf = jnp.full((16,), pw.astype(jnp.float32))
                x = sw + pwf * bw + (ixf + 0.5) * bwh
                x = jnp.maximum(x, 0.0)
                xl0 = x.astype(jnp.int32)
                xh = jnp.where(xl0 >= wm1, wm1, xl0 + 1)
                xl = jnp.where(xl0 >= wm1, wm1, xl0)
                lx = x - xl.astype(jnp.float32)
                wx = jnp.where(cx1, lx, 1.0 - lx)
                xsel = jnp.where(cx1, xh, xl)
                o = (bx * (PH * PW) + ph * PW + pw) * K
                idx_v[pl.ds(o, K)] = yterm + xsel
                w_v[pl.ds(o, K)] = wy * wx * 0.25
                return c2

            lax.fori_loop(0, PW, pw_loop, 0)
            return c1

        lax.fori_loop(0, PH, ph_loop, 0)


    def issue(i, b):
        idx_vec = idx_v[pl.ds(i * K, K)]
        pltpu.async_copy(table.at[idx_vec], gbuf.at[b], sem.at[b])

    def drain(b):
        pltpu.make_async_copy(table.at[pl.ds(0, K)], gbuf.at[b],
                              sem.at[b]).wait()

    def odrain(h):
        pltpu.make_async_copy(obuf.at[h], out.at[pl.ds(base, CHUNK)],
                              osem.at[h]).wait()

    for b in range(NBUF):
        issue(b, b)

    def chunk_loop(cix, carry):
        h = lax.rem(cix, 2)

        @pl.when(cix >= 2)
        def _wait_prev_flush():
            odrain(h)

        def group(gg, carry2):
            g = cix * GPC + gg
            for b in range(NBUF):
                i = g * NBUF + b
                local = gg * NBUF + b
                drain(b)
                w_vec = w_v[pl.ds(i * K, K)]
                acc = [jnp.zeros((16,), jnp.float32) for _ in range(CCHUNKS)]
                for k in range(K):
                    wk = _lane_bcast(w_vec, k)
                    for j in range(CCHUNKS):
                        acc[j] = acc[j] + wk * gbuf[b, k, pl.ds(16 * j, 16)]
                for j in range(CCHUNKS):
                    obuf[h, local, pl.ds(16 * j, 16)] = acc[j]

                @pl.when(g < NGROUPS - 1)
                def _issue_next():
                    issue(i + NBUF, b)
            return carry2

        lax.fori_loop(0, GPC, group, 0)
        pltpu.async_copy(obuf.at[h], out.at[pl.ds(base + cix * CHUNK, CHUNK)],
                         osem.at[h])
        return carry

    lax.fori_loop(0, NCHUNK, chunk_loop, 0)
    odrain(1)
    odrain(0)


@jax.jit
def _sc_gather_interp(table, params):
    mesh = plsc.VectorSubcoreMesh(core_axis_name="c", subcore_axis_name="s")
    f = functools.partial(
        pl.kernel, mesh=mesh,
        out_type=jax.ShapeDtypeStruct((BINS, C), jnp.float32),
        scratch_types=[
            pltpu.VMEM((ROWS_PER_W * K,), jnp.int32),
            pltpu.VMEM((ROWS_PER_W * K,), jnp.float32),
            pltpu.VMEM((NBUF, K, C), jnp.float32),
            pltpu.VMEM((2, CHUNK, C), jnp.float32),
            pltpu.VMEM((BOXES_PER_W, 16), jnp.float32),
            pltpu.SemaphoreType.DMA((NBUF,)),
            pltpu.SemaphoreType.DMA((2,)),
        ],
    )(_sc_body)
    return f(table, params)


def _box_params(boxes):
    f32 = jnp.float32
    area = (boxes[:, 2] - boxes[:, 0]) * (boxes[:, 3] - boxes[:, 1])
    s = jnp.sqrt(area)
    target_lvls = jnp.floor(4.0 + jnp.log2(s / 224.0) + 1e-6)
    lvl = jnp.clip(target_lvls, 2.0, 5.0).astype(jnp.int32) - 2
    scale = jnp.take(jnp.array(_SCALES, f32), lvl)
    wf = jnp.take(jnp.array(_WS, f32), lvl)
    hf = jnp.take(jnp.array(_HS, f32), lvl)
    basef = jnp.take(jnp.array(_BASES, f32), lvl)
    start_w = boxes[:, 0] * scale
    start_h = boxes[:, 1] * scale
    end_w = boxes[:, 2] * scale
    end_h = boxes[:, 3] * scale
    roi_w = jnp.maximum(end_w - start_w, 1.0)
    roi_h = jnp.maximum(end_h - start_h, 1.0)
    bin_h = roi_h / PH
    bin_w = roi_w / PW
    z = jnp.zeros_like(scale)
    cols = [start_h, start_w, bin_h, bin_w, bin_h / SR, bin_w / SR,
            wf, hf - 1.0, wf - 1.0, basef, z, z, z, z, z, z]
    return jnp.stack(cols, axis=1)


def kernel(feat0, feat1, feat2, feat3, boxes):
    table = jnp.concatenate(
        [f[0].transpose(1, 2, 0).reshape(-1, C)
         for f in (feat0, feat1, feat2, feat3)], axis=0)
    params = _box_params(boxes)
    res = _sc_gather_interp(table, params)
    return res.reshape(NBOX, PH * PW, C).transpose(0, 2, 1).reshape(NBOX, C, PH, PW)
